# baseline (speedup 1.0000x reference)
import functools

import jax
import jax.numpy as jnp
from jax import lax
from jax.experimental import pallas as pl
from jax.experimental.pallas import tpu as pltpu
from jax.experimental.pallas import tpu_sc as plsc

B = 16384
D = 64
_EG = 0.5772156649015329
_TINY = 1.1754943508222875e-38


_NE = 100000
_CB = 8192
_NBLK = 13
_NEP = _NBLK * _CB


def _repack_body(*refs):
    n = len(refs) // 2
    ins, outs = refs[:n], refs[n:]
    for tt_r, out_r in zip(ins, outs):
        x = tt_r[...].astype(jnp.bfloat16)
        ta = jnp.transpose(x[:, :_CB // 2])
        tb = jnp.transpose(x[:, _CB // 2:])
        out_r[...] = jnp.concatenate([ta, tb], axis=1)


def _tc_repack(*tts):
    n = len(tts)
    out = pl.pallas_call(
        _repack_body,
        grid=(_NBLK,),
        in_specs=[pl.BlockSpec((D, _CB), lambda i: (0, i))] * n,
        out_specs=[pl.BlockSpec((_CB // 2, 2 * D), lambda i: (i, 0))] * n,
        out_shape=[jax.ShapeDtypeStruct((_NEP // 2, 2 * D), jnp.bfloat16)] * n,
    )(*tts)
    return [o.reshape(-1).reshape(_NEP, D) for o in out]


_SC_NC = 2
_SC_NS = 16
_CH = 128


_HB = _CB // 2
_HBITS = _HB.bit_length() - 1


def _permute_idx(idx_ref):
    for m in range(_CH // 16):
        e = idx_ref[pl.ds(m * 16, 16)]
        r = ((e & jnp.int32(~(_CB - 1)))
             | ((e & jnp.int32(_HB - 1)) << 1)
             | ((e >> _HBITS) & jnp.int32(1)))
        idx_ref[pl.ds(m * 16, 16)] = r


@functools.lru_cache(maxsize=None)
def _make_sc_gather_ent():
    mesh = plsc.VectorSubcoreMesh(core_axis_name="c", subcore_axis_name="s")

    @functools.partial(
        pl.kernel,
        mesh=mesh,
        out_type=[jax.ShapeDtypeStruct((B, D), jnp.bfloat16)] * 4,
        scratch_types=[
            pltpu.VMEM((_CH,), jnp.int32),
            pltpu.VMEM((_CH,), jnp.int32),
        ]
        + [pltpu.VMEM((_CH, D), jnp.bfloat16) for _ in range(4)]
        + [pltpu.SemaphoreType.DMA, pltpu.SemaphoreType.DMA],
        compiler_params=pltpu.CompilerParams(use_tc_tiling_on_sc=False),
    )
    def sc_gather_ent(heads, tails, min_e, dl_e,
                      o_hmin, o_hdl, o_tmin, o_tdl,
                      hidx, tidx, b0, b1, b2, b3, gsem, wsem):
        nw = _SC_NC * _SC_NS
        n_per = B // nw
        wid = lax.axis_index("s") * _SC_NC + lax.axis_index("c")
        base = wid * n_per
        tabs = (min_e, dl_e, min_e, dl_e)
        idxs = (hidx, hidx, tidx, tidx)
        bufs = (b0, b1, b2, b3)
        outs = (o_hmin, o_hdl, o_tmin, o_tdl)
        for c in range(n_per // _CH):
            co = base + c * _CH
            pltpu.sync_copy(heads.at[pl.ds(co, _CH)], hidx)
            pltpu.sync_copy(tails.at[pl.ds(co, _CH)], tidx)
            _permute_idx(hidx)
            _permute_idx(tidx)
            gathers = [
                pltpu.async_copy(tabs[j].at[idxs[j]], bufs[j], gsem)
                for j in range(4)
            ]
            wbs = []
            for j in range(4):
                gathers[j].wait()
                wbs.append(
                    pltpu.async_copy(bufs[j], outs[j].at[pl.ds(co, _CH)], wsem)
                )
            for w in wbs:
                w.wait()

    return sc_gather_ent


@functools.lru_cache(maxsize=None)
def _make_sc_gather_rel():
    mesh = plsc.VectorSubcoreMesh(core_axis_name="c", subcore_axis_name="s")

    @functools.partial(
        pl.kernel,
        mesh=mesh,
        out_type=[jax.ShapeDtypeStruct((B, D), jnp.bfloat16)] * 4,
        scratch_types=[
            pltpu.VMEM((_CH,), jnp.int32),
        ]
        + [pltpu.VMEM((_CH, D), jnp.bfloat16) for _ in range(4)]
        + [pltpu.SemaphoreType.DMA, pltpu.SemaphoreType.DMA],
        compiler_params=pltpu.CompilerParams(use_tc_tiling_on_sc=False),
    )
    def sc_gather_rel(rels, trh, sch, trt, sct,
                      o_trh, o_sch, o_trt, o_sct,
                      ridx, b0, b1, b2, b3, gsem, wsem):
        nw = _SC_NC * _SC_NS
        n_per = B // nw
        wid = lax.axis_index("s") * _SC_NC + lax.axis_index("c")
        base = wid * n_per
        tabs = (trh, sch, trt, sct)
        bufs = (b0, b1, b2, b3)
        outs = (o_trh, o_sch, o_trt, o_sct)
        for c in range(n_per // _CH):
            co = base + c * _CH
            pltpu.sync_copy(rels.at[pl.ds(co, _CH)], ridx)
            _permute_idx(ridx)
            gathers = [
                pltpu.async_copy(tabs[j].at[ridx], bufs[j], gsem)
                for j in range(4)
            ]
            wbs = []
            for j in range(4):
                gathers[j].wait()
                wbs.append(
                    pltpu.async_copy(bufs[j], outs[j].at[pl.ds(co, _CH)], wsem)
                )
            for w in wbs:
                w.wait()

    return sc_gather_rel


_TC_R = 1024


def _tc_body(hmin_r, hdl_r, tmin_r, tdl_r, trh_r, sch_r, trt_r, sct_r,
             ts_r, te_r, w1_r, b1_r, w2c0_r, w2c1_r, w2c2_r, b2_r, out_r):
    lanes = lax.broadcasted_iota(jnp.int32, (1, 2 * D), 1)
    left = lanes < D

    def mlp_time(ts1):
        h = jnp.maximum(ts1 * w1_r[...] + b1_r[...], 0.0)
        z = (b2_r[...] + h[:, 0:1] * w2c0_r[...] + h[:, 1:2] * w2c1_r[...]
             + h[:, 2:3] * w2c2_r[...])
        td = 1.0 / (1.0 + jnp.exp(-z))
        te = te_r[...]
        return (td[:, 0:1] * te[0:1, :] + td[:, 1:2] * te[1:2, :]
                + td[:, 2:3] * te[2:3, :])

    tm = jnp.concatenate([mlp_time(ts_r[:, 0:1]), mlp_time(ts_r[:, 1:2])],
                         axis=1)

    def hsums(x):
        se = jnp.sum(x[:, :D], axis=1, keepdims=True)
        so = jnp.sum(x[:, D:], axis=1, keepdims=True)
        return jnp.where(left, se, so)

    def transform(mn, dl, tr, sc):
        trp = tr - tm * hsums(tr * tm)
        scp = sc - tm * hsums(sc * tm)
        mn2 = mn + trp
        dl2 = dl * scp
        return mn2, dl2, mn2 + dl2

    f32 = lambda r: r[...].astype(jnp.float32)
    hmn2, hdl2, hmx2 = transform(f32(hmin_r), jnp.exp(f32(hdl_r)),
                                 f32(trh_r), f32(sch_r))
    tmn2, tdl2, tmx2 = transform(f32(tmin_r), jnp.exp(f32(tdl_r)),
                                 f32(trt_r), f32(sct_r))

    def lae(a, b):
        return jnp.maximum(a, b) + jnp.log1p(jnp.exp(-jnp.abs(a - b)))

    imn = jnp.maximum(lae(hmn2, tmn2), jnp.maximum(hmn2, tmn2))
    imx = jnp.minimum(-lae(-hmx2, -tmx2), jnp.minimum(hmx2, tmx2))

    c2g = 2.0 * _EG

    def log_vol(d):
        x = d - c2g
        sp = jnp.maximum(x, 0.0) + jnp.log1p(jnp.exp(-jnp.abs(x)))
        sp = jnp.maximum(sp, _TINY)
        l = jnp.log(sp)
        return (jnp.sum(l[:, :D], axis=1, keepdims=True),
                jnp.sum(l[:, D:], axis=1, keepdims=True))

    li_e, li_o = log_vol(imx - imn)
    lh_e, lh_o = log_vol(hdl2)
    lt_e, lt_o = log_vol(tdl2)
    pe = jnp.minimum(li_e - lh_e, li_e - lt_e)
    po = jnp.minimum(li_o - lh_o, li_o - lt_o)
    out_r[...] = jnp.concatenate([pe, po], axis=1)


def _tc_math(hmin, hdl, tmin, tdl, trh, sch, trt, sct, ts2, te,
             w1r, b1r, w2c0, w2c1, w2c2, b2r):
    grid = (B // 2 // _TC_R,)
    row = pl.BlockSpec((_TC_R, 2 * D), lambda i: (i, 0))
    two = pl.BlockSpec((_TC_R, 2), lambda i: (i, 0))
    small3 = pl.BlockSpec((1, 3), lambda i: (0, 0))
    tes = pl.BlockSpec((3, D), lambda i: (0, 0))
    return pl.pallas_call(
        _tc_body,
        grid=grid,
        in_specs=[row] * 8 + [two, tes, small3, small3, small3, small3,
                              small3, small3],
        out_specs=two,
        out_shape=jax.ShapeDtypeStruct((B // 2, 2), jnp.float32),
    )(hmin, hdl, tmin, tdl, trh, sch, trt, sct, ts2, te,
      w1r, b1r, w2c0, w2c1, w2c2, b2r)


def kernel(samples, min_embedding, delta_embedding, time_embedding,
           W1, b1, W2, b2, rel_trans_for_head, rel_scale_for_head,
           rel_trans_for_tail, rel_scale_for_tail):
    heads = samples[:, 0]
    tails = samples[:, 1]
    rels = samples[:, 2]
    ts2 = samples[:, 3].astype(jnp.float32).reshape(B // 2, 2)

    ent_tabs = _tc_repack(min_embedding.T, delta_embedding.T)
    g_ent = _make_sc_gather_ent()(heads, tails, *ent_tabs)
    rel_tabs = _tc_repack(rel_trans_for_head.T, rel_scale_for_head.T,
                          rel_trans_for_tail.T, rel_scale_for_tail.T)
    g_rel = _make_sc_gather_rel()(rels, *rel_tabs)
    g = list(g_ent) + list(g_rel)
    gp = [x.reshape(-1).reshape(B // 2, 2 * D) for x in g]

    w1r = W1.reshape(1, 3)
    b1r = b1.reshape(1, 3)
    w2c0 = W2[:, 0].reshape(1, 3)
    w2c1 = W2[:, 1].reshape(1, 3)
    w2c2 = W2[:, 2].reshape(1, 3)
    b2r = b2.reshape(1, 3)

    out = _tc_math(*gp, ts2, time_embedding, w1r, b1r, w2c0, w2c1, w2c2, b2r)
    return out.reshape(B)


# speedup vs baseline: 2.2145x; 2.2145x over previous
import functools

import jax
import jax.numpy as jnp
from jax import lax
from jax.experimental import pallas as pl
from jax.experimental.pallas import tpu as pltpu
from jax.experimental.pallas import tpu_sc as plsc

B = 16384
D = 64
_EG = 0.5772156649015329
_TINY = 1.1754943508222875e-38


_NE = 100000
_CB = 8192
_NBLK = 13
_NEP = _NBLK * _CB


def _repack_body(*refs):
    n = len(refs) // 2
    ins, outs = refs[:n], refs[n:]
    for tt_r, out_r in zip(ins, outs):
        x = tt_r[...]
        ta = jnp.transpose(x[:, :_CB // 2])
        tb = jnp.transpose(x[:, _CB // 2:])
        out_r[...] = jnp.concatenate([ta, tb], axis=1)


def _tc_repack(*tts):
    n = len(tts)
    out = pl.pallas_call(
        _repack_body,
        grid=(_NBLK,),
        in_specs=[pl.BlockSpec((D, _CB), lambda i: (0, i))] * n,
        out_specs=[pl.BlockSpec((_CB // 2, 2 * D), lambda i: (i, 0))] * n,
        out_shape=[jax.ShapeDtypeStruct((_NEP // 2, 2 * D), jnp.float32)] * n,
    )(*tts)
    return [o.reshape(-1).reshape(_NEP, D) for o in out]


_SC_NC = 2
_SC_NS = 16
_CH = 128


_HB = _CB // 2
_HBITS = _HB.bit_length() - 1


def _permute_idx(idx_ref):
    for m in range(_CH // 16):
        e = idx_ref[pl.ds(m * 16, 16)]
        r = ((e & jnp.int32(~(_CB - 1)))
             | ((e & jnp.int32(_HB - 1)) << 1)
             | ((e >> _HBITS) & jnp.int32(1)))
        idx_ref[pl.ds(m * 16, 16)] = r


@functools.lru_cache(maxsize=None)
def _make_sc_gather_ent():
    mesh = plsc.VectorSubcoreMesh(core_axis_name="c", subcore_axis_name="s")

    @functools.partial(
        pl.kernel,
        mesh=mesh,
        out_type=[jax.ShapeDtypeStruct((B, D), jnp.float32)] * 4,
        scratch_types=[
            pltpu.VMEM((_CH,), jnp.int32),
            pltpu.VMEM((_CH,), jnp.int32),
        ]
        + [pltpu.VMEM((_CH, D), jnp.float32) for _ in range(4)]
        + [pltpu.SemaphoreType.DMA, pltpu.SemaphoreType.DMA],
        compiler_params=pltpu.CompilerParams(use_tc_tiling_on_sc=False),
    )
    def sc_gather_ent(heads, tails, min_e, dl_e,
                      o_hmin, o_hdl, o_tmin, o_tdl,
                      hidx, tidx, b0, b1, b2, b3, gsem, wsem):
        nw = _SC_NC * _SC_NS
        n_per = B // nw
        wid = lax.axis_index("s") * _SC_NC + lax.axis_index("c")
        base = wid * n_per
        tabs = (min_e, dl_e, min_e, dl_e)
        idxs = (hidx, hidx, tidx, tidx)
        bufs = (b0, b1, b2, b3)
        outs = (o_hmin, o_hdl, o_tmin, o_tdl)
        for c in range(n_per // _CH):
            co = base + c * _CH
            pltpu.sync_copy(heads.at[pl.ds(co, _CH)], hidx)
            pltpu.sync_copy(tails.at[pl.ds(co, _CH)], tidx)
            _permute_idx(hidx)
            _permute_idx(tidx)
            gathers = [
                pltpu.async_copy(tabs[j].at[idxs[j]], bufs[j], gsem)
                for j in range(4)
            ]
            wbs = []
            for j in range(4):
                gathers[j].wait()
                wbs.append(
                    pltpu.async_copy(bufs[j], outs[j].at[pl.ds(co, _CH)], wsem)
                )
            for w in wbs:
                w.wait()

    return sc_gather_ent


@functools.lru_cache(maxsize=None)
def _make_sc_gather_rel():
    mesh = plsc.VectorSubcoreMesh(core_axis_name="c", subcore_axis_name="s")

    @functools.partial(
        pl.kernel,
        mesh=mesh,
        out_type=[jax.ShapeDtypeStruct((B, D), jnp.float32)] * 4,
        scratch_types=[
            pltpu.VMEM((_CH,), jnp.int32),
        ]
        + [pltpu.VMEM((_CH, D), jnp.float32) for _ in range(4)]
        + [pltpu.SemaphoreType.DMA, pltpu.SemaphoreType.DMA],
        compiler_params=pltpu.CompilerParams(use_tc_tiling_on_sc=False),
    )
    def sc_gather_rel(rels, trh, sch, trt, sct,
                      o_trh, o_sch, o_trt, o_sct,
                      ridx, b0, b1, b2, b3, gsem, wsem):
        nw = _SC_NC * _SC_NS
        n_per = B // nw
        wid = lax.axis_index("s") * _SC_NC + lax.axis_index("c")
        base = wid * n_per
        tabs = (trh, sch, trt, sct)
        bufs = (b0, b1, b2, b3)
        outs = (o_trh, o_sch, o_trt, o_sct)
        for c in range(n_per // _CH):
            co = base + c * _CH
            pltpu.sync_copy(rels.at[pl.ds(co, _CH)], ridx)
            _permute_idx(ridx)
            gathers = [
                pltpu.async_copy(tabs[j].at[ridx], bufs[j], gsem)
                for j in range(4)
            ]
            wbs = []
            for j in range(4):
                gathers[j].wait()
                wbs.append(
                    pltpu.async_copy(bufs[j], outs[j].at[pl.ds(co, _CH)], wsem)
                )
            for w in wbs:
                w.wait()

    return sc_gather_rel


_TC_R = 1024


def _tc_body(hmin_r, hdl_r, tmin_r, tdl_r, trh_r, sch_r, trt_r, sct_r,
             ts_r, te_r, w1_r, b1_r, w2c0_r, w2c1_r, w2c2_r, b2_r, out_r):
    lanes = lax.broadcasted_iota(jnp.int32, (1, 2 * D), 1)
    left = lanes < D

    def mlp_time(ts1):
        h = jnp.maximum(ts1 * w1_r[...] + b1_r[...], 0.0)
        z = (b2_r[...] + h[:, 0:1] * w2c0_r[...] + h[:, 1:2] * w2c1_r[...]
             + h[:, 2:3] * w2c2_r[...])
        td = 1.0 / (1.0 + jnp.exp(-z))
        te = te_r[...]
        return (td[:, 0:1] * te[0:1, :] + td[:, 1:2] * te[1:2, :]
                + td[:, 2:3] * te[2:3, :])

    tm = jnp.concatenate([mlp_time(ts_r[:, 0:1]), mlp_time(ts_r[:, 1:2])],
                         axis=1)

    def hsums(x):
        se = jnp.sum(x[:, :D], axis=1, keepdims=True)
        so = jnp.sum(x[:, D:], axis=1, keepdims=True)
        return jnp.where(left, se, so)

    def transform(mn, dl, tr, sc):
        trp = tr - tm * hsums(tr * tm)
        scp = sc - tm * hsums(sc * tm)
        mn2 = mn + trp
        dl2 = dl * scp
        return mn2, dl2, mn2 + dl2

    hmn2, hdl2, hmx2 = transform(hmin_r[...], jnp.exp(hdl_r[...]),
                                 trh_r[...], sch_r[...])
    tmn2, tdl2, tmx2 = transform(tmin_r[...], jnp.exp(tdl_r[...]),
                                 trt_r[...], sct_r[...])

    def lae(a, b):
        return jnp.maximum(a, b) + jnp.log1p(jnp.exp(-jnp.abs(a - b)))

    imn = jnp.maximum(lae(hmn2, tmn2), jnp.maximum(hmn2, tmn2))
    imx = jnp.minimum(-lae(-hmx2, -tmx2), jnp.minimum(hmx2, tmx2))

    c2g = 2.0 * _EG

    def log_vol(d):
        x = d - c2g
        sp = jnp.maximum(x, 0.0) + jnp.log1p(jnp.exp(-jnp.abs(x)))
        sp = jnp.maximum(sp, _TINY)
        l = jnp.log(sp)
        return (jnp.sum(l[:, :D], axis=1, keepdims=True),
                jnp.sum(l[:, D:], axis=1, keepdims=True))

    li_e, li_o = log_vol(imx - imn)
    lh_e, lh_o = log_vol(hdl2)
    lt_e, lt_o = log_vol(tdl2)
    pe = jnp.minimum(li_e - lh_e, li_e - lt_e)
    po = jnp.minimum(li_o - lh_o, li_o - lt_o)
    out_r[...] = jnp.concatenate([pe, po], axis=1)


def _tc_math(hmin, hdl, tmin, tdl, trh, sch, trt, sct, ts2, te,
             w1r, b1r, w2c0, w2c1, w2c2, b2r):
    grid = (B // 2 // _TC_R,)
    row = pl.BlockSpec((_TC_R, 2 * D), lambda i: (i, 0))
    two = pl.BlockSpec((_TC_R, 2), lambda i: (i, 0))
    small3 = pl.BlockSpec((1, 3), lambda i: (0, 0))
    tes = pl.BlockSpec((3, D), lambda i: (0, 0))
    return pl.pallas_call(
        _tc_body,
        grid=grid,
        in_specs=[row] * 8 + [two, tes, small3, small3, small3, small3,
                              small3, small3],
        out_specs=two,
        out_shape=jax.ShapeDtypeStruct((B // 2, 2), jnp.float32),
    )(hmin, hdl, tmin, tdl, trh, sch, trt, sct, ts2, te,
      w1r, b1r, w2c0, w2c1, w2c2, b2r)


def kernel(samples, min_embedding, delta_embedding, time_embedding,
           W1, b1, W2, b2, rel_trans_for_head, rel_scale_for_head,
           rel_trans_for_tail, rel_scale_for_tail):
    heads = samples[:, 0]
    tails = samples[:, 1]
    rels = samples[:, 2]
    ts2 = samples[:, 3].astype(jnp.float32).reshape(B // 2, 2)

    ent_tabs = _tc_repack(min_embedding.T, delta_embedding.T)
    g_ent = _make_sc_gather_ent()(heads, tails, *ent_tabs)
    rel_tabs = _tc_repack(rel_trans_for_head.T, rel_scale_for_head.T,
                          rel_trans_for_tail.T, rel_scale_for_tail.T)
    g_rel = _make_sc_gather_rel()(rels, *rel_tabs)
    g = list(g_ent) + list(g_rel)
    gp = [x.reshape(-1).reshape(B // 2, 2 * D) for x in g]

    w1r = W1.reshape(1, 3)
    b1r = b1.reshape(1, 3)
    w2c0 = W2[:, 0].reshape(1, 3)
    w2c1 = W2[:, 1].reshape(1, 3)
    w2c2 = W2[:, 2].reshape(1, 3)
    b2r = b2.reshape(1, 3)

    out = _tc_math(*gp, ts2, time_embedding, w1r, b1r, w2c0, w2c1, w2c2, b2r)
    return out.reshape(B)


# speedup vs baseline: 2.2925x; 1.0352x over previous
import functools

import jax
import jax.numpy as jnp
from jax import lax
from jax.experimental import pallas as pl
from jax.experimental.pallas import tpu as pltpu
from jax.experimental.pallas import tpu_sc as plsc

B = 16384
D = 64
_EG = 0.5772156649015329
_TINY = 1.1754943508222875e-38


_NE = 100000
_CB = 4096
_NBLK = 25
_NEP = _NBLK * _CB


def _repack_body(*refs):
    n = len(refs) // 3
    for k in range(n):
        xa, xb, o = refs[2 * k], refs[2 * k + 1], refs[2 * n + k]
        x2 = jnp.concatenate([xa[...], xb[...]], axis=0)
        o[...] = jnp.transpose(x2)


def _tc_repack(*tabs):
    n = len(tabs) // 2
    outs = pl.pallas_call(
        _repack_body,
        grid=(_NBLK,),
        in_specs=[pl.BlockSpec((D, _CB), lambda i: (0, i))] * (2 * n),
        out_specs=[pl.BlockSpec((_CB, 2 * D), lambda i: (i, 0))] * n,
        out_shape=[jax.ShapeDtypeStruct((_NEP, 2 * D), jnp.float32)] * n,
    )(*[t.T for t in tabs])
    return list(outs)


_SC_NC = 2
_SC_NS = 16
_CH = 128


@functools.lru_cache(maxsize=None)
def _make_sc_gather_ent():
    mesh = plsc.VectorSubcoreMesh(core_axis_name="c", subcore_axis_name="s")

    @functools.partial(
        pl.kernel,
        mesh=mesh,
        out_type=[jax.ShapeDtypeStruct((B, 2 * D), jnp.float32)] * 2,
        scratch_types=[
            pltpu.VMEM((_CH,), jnp.int32),
            pltpu.VMEM((_CH,), jnp.int32),
            pltpu.VMEM((_CH, 2 * D), jnp.float32),
            pltpu.VMEM((_CH, 2 * D), jnp.float32),
            pltpu.SemaphoreType.DMA,
            pltpu.SemaphoreType.DMA,
        ],
        compiler_params=pltpu.CompilerParams(use_tc_tiling_on_sc=False),
    )
    def sc_gather_ent(heads, tails, ent_pair,
                      o_h, o_t, hidx, tidx, b0, b1, gsem, wsem):
        nw = _SC_NC * _SC_NS
        n_per = B // nw
        wid = lax.axis_index("s") * _SC_NC + lax.axis_index("c")
        base = wid * n_per
        for c in range(n_per // _CH):
            co = base + c * _CH
            pltpu.sync_copy(heads.at[pl.ds(co, _CH)], hidx)
            pltpu.sync_copy(tails.at[pl.ds(co, _CH)], tidx)
            g0 = pltpu.async_copy(ent_pair.at[hidx], b0, gsem)
            g1 = pltpu.async_copy(ent_pair.at[tidx], b1, gsem)
            g0.wait()
            w0 = pltpu.async_copy(b0, o_h.at[pl.ds(co, _CH)], wsem)
            g1.wait()
            w1 = pltpu.async_copy(b1, o_t.at[pl.ds(co, _CH)], wsem)
            w0.wait()
            w1.wait()

    return sc_gather_ent


@functools.lru_cache(maxsize=None)
def _make_sc_gather_rel():
    mesh = plsc.VectorSubcoreMesh(core_axis_name="c", subcore_axis_name="s")

    @functools.partial(
        pl.kernel,
        mesh=mesh,
        out_type=[jax.ShapeDtypeStruct((B, 2 * D), jnp.float32)] * 2,
        scratch_types=[
            pltpu.VMEM((_CH,), jnp.int32),
            pltpu.VMEM((_CH, 2 * D), jnp.float32),
            pltpu.VMEM((_CH, 2 * D), jnp.float32),
            pltpu.SemaphoreType.DMA,
            pltpu.SemaphoreType.DMA,
        ],
        compiler_params=pltpu.CompilerParams(use_tc_tiling_on_sc=False),
    )
    def sc_gather_rel(rels, rh_pair, rt_pair,
                      o_rh, o_rt, ridx, b0, b1, gsem, wsem):
        nw = _SC_NC * _SC_NS
        n_per = B // nw
        wid = lax.axis_index("s") * _SC_NC + lax.axis_index("c")
        base = wid * n_per
        for c in range(n_per // _CH):
            co = base + c * _CH
            pltpu.sync_copy(rels.at[pl.ds(co, _CH)], ridx)
            g0 = pltpu.async_copy(rh_pair.at[ridx], b0, gsem)
            g1 = pltpu.async_copy(rt_pair.at[ridx], b1, gsem)
            g0.wait()
            w0 = pltpu.async_copy(b0, o_rh.at[pl.ds(co, _CH)], wsem)
            g1.wait()
            w1 = pltpu.async_copy(b1, o_rt.at[pl.ds(co, _CH)], wsem)
            w0.wait()
            w1.wait()

    return sc_gather_rel


_TC_R = 2048


def _tc_body(he_r, te_r2, rh_r, rt_r,
             ts_r, te_r, w1_r, b1_r, w2c0_r, w2c1_r, w2c2_r, b2_r, out_r):
    he = he_r[...]
    tt = te_r2[...]
    rh = rh_r[...]
    rt = rt_r[...]

    ts1 = ts_r[...]
    h = jnp.maximum(ts1 * w1_r[...] + b1_r[...], 0.0)
    z = (b2_r[...] + h[:, 0:1] * w2c0_r[...] + h[:, 1:2] * w2c1_r[...]
         + h[:, 2:3] * w2c2_r[...])
    td = 1.0 / (1.0 + jnp.exp(-z))
    te = te_r[...]
    time = (td[:, 0:1] * te[0:1, :] + td[:, 1:2] * te[1:2, :]
            + td[:, 2:3] * te[2:3, :])

    def transform(mn, dl, tr, sc):
        trp = tr - time * jnp.sum(tr * time, axis=1, keepdims=True)
        scp = sc - time * jnp.sum(sc * time, axis=1, keepdims=True)
        mn2 = mn + trp
        dl2 = dl * scp
        return mn2, dl2, mn2 + dl2

    hmn2, hdl2, hmx2 = transform(he[:, :D], jnp.exp(he[:, D:]),
                                 rh[:, :D], rh[:, D:])
    tmn2, tdl2, tmx2 = transform(tt[:, :D], jnp.exp(tt[:, D:]),
                                 rt[:, :D], rt[:, D:])

    def lae(a, b):
        return jnp.maximum(a, b) + jnp.log1p(jnp.exp(-jnp.abs(a - b)))

    imn = jnp.maximum(lae(hmn2, tmn2), jnp.maximum(hmn2, tmn2))
    imx = jnp.minimum(-lae(-hmx2, -tmx2), jnp.minimum(hmx2, tmx2))

    c2g = 2.0 * _EG

    def log_vol(d):
        x = d - c2g
        sp = jnp.maximum(x, 0.0) + jnp.log1p(jnp.exp(-jnp.abs(x)))
        sp = jnp.maximum(sp, _TINY)
        return jnp.sum(jnp.log(sp), axis=1, keepdims=True)

    li = log_vol(imx - imn)
    lh = log_vol(hdl2)
    lt = log_vol(tdl2)
    out_r[...] = jnp.minimum(li - lh, li - lt)


def _tc_math(he, te2, rh, rt, ts, te, w1r, b1r, w2c0, w2c1, w2c2, b2r):
    grid = (B // _TC_R,)
    row = pl.BlockSpec((_TC_R, 2 * D), lambda i: (i, 0))
    one = pl.BlockSpec((_TC_R, 1), lambda i: (i, 0))
    small3 = pl.BlockSpec((1, 3), lambda i: (0, 0))
    tes = pl.BlockSpec((3, D), lambda i: (0, 0))
    return pl.pallas_call(
        _tc_body,
        grid=grid,
        in_specs=[row] * 4 + [one, tes, small3, small3, small3, small3,
                              small3, small3],
        out_specs=one,
        out_shape=jax.ShapeDtypeStruct((B, 1), jnp.float32),
    )(he, te2, rh, rt, ts, te, w1r, b1r, w2c0, w2c1, w2c2, b2r)


def kernel(samples, min_embedding, delta_embedding, time_embedding,
           W1, b1, W2, b2, rel_trans_for_head, rel_scale_for_head,
           rel_trans_for_tail, rel_scale_for_tail):
    heads = samples[:, 0]
    tails = samples[:, 1]
    rels = samples[:, 2]
    ts = samples[:, 3].astype(jnp.float32)[:, None]

    (ent_pair,) = _tc_repack(min_embedding, delta_embedding)
    g_h, g_t = _make_sc_gather_ent()(heads, tails, ent_pair)
    rh_pair, rt_pair = _tc_repack(rel_trans_for_head, rel_scale_for_head,
                                  rel_trans_for_tail, rel_scale_for_tail)
    g_rh, g_rt = _make_sc_gather_rel()(rels, rh_pair, rt_pair)

    w1r = W1.reshape(1, 3)
    b1r = b1.reshape(1, 3)
    w2c0 = W2[:, 0].reshape(1, 3)
    w2c1 = W2[:, 1].reshape(1, 3)
    w2c2 = W2[:, 2].reshape(1, 3)
    b2r = b2.reshape(1, 3)

    out = _tc_math(g_h, g_t, g_rh, g_rt, ts, time_embedding,
                   w1r, b1r, w2c0, w2c1, w2c2, b2r)
    return out[:, 0]
